# Initial kernel scaffold; baseline (speedup 1.0000x reference)
import jax
import jax.numpy as jnp
from jax.experimental import pallas as pl


def kernel(x, W_gate, W_up, W_down):
    raise NotImplementedError("write your pallas kernel here")


# speedup vs baseline: 1.1510x; 1.1510x over previous
import jax
import jax.numpy as jnp
from jax.experimental import pallas as pl

_HIDDEN = 2048
_INTER = 2048
_TOKENS = 32
_BJ = 512


def _mlp_kernel(x_ref, wg_ref, wu_ref, wd_ref, o_ref):
    j = pl.program_id(0)
    x = x_ref[...]
    gate = jnp.dot(x, wg_ref[...], preferred_element_type=jnp.float32)
    up = jnp.dot(x, wu_ref[...], preferred_element_type=jnp.float32)
    act = gate * jax.nn.sigmoid(gate) * up
    contrib = jnp.dot(act, wd_ref[...], preferred_element_type=jnp.float32)

    @pl.when(j == 0)
    def _init():
        o_ref[...] = contrib

    @pl.when(j > 0)
    def _acc():
        o_ref[...] += contrib


def kernel(x, W_gate, W_up, W_down):
    return pl.pallas_call(
        _mlp_kernel,
        grid=(_INTER // _BJ,),
        in_specs=[
            pl.BlockSpec((_TOKENS, _HIDDEN), lambda j: (0, 0)),
            pl.BlockSpec((_HIDDEN, _BJ), lambda j: (0, j)),
            pl.BlockSpec((_HIDDEN, _BJ), lambda j: (0, j)),
            pl.BlockSpec((_BJ, _HIDDEN), lambda j: (j, 0)),
        ],
        out_specs=pl.BlockSpec((_TOKENS, _HIDDEN), lambda j: (0, 0)),
        out_shape=jax.ShapeDtypeStruct((_TOKENS, _HIDDEN), jnp.float32),
    )(x, W_gate, W_up, W_down)


# speedup vs baseline: 1.1741x; 1.0201x over previous
import jax
import jax.numpy as jnp
from jax.experimental import pallas as pl

_HIDDEN = 2048
_INTER = 2048
_TOKENS = 32
_BJ = 256


def _mlp_kernel(x_ref, wg_ref, wu_ref, wd_ref, o_ref):
    j = pl.program_id(0)
    x = x_ref[...]
    gate = jnp.dot(x, wg_ref[...], preferred_element_type=jnp.float32)
    up = jnp.dot(x, wu_ref[...], preferred_element_type=jnp.float32)
    act = gate * jax.nn.sigmoid(gate) * up
    contrib = jnp.dot(act, wd_ref[...], preferred_element_type=jnp.float32)

    @pl.when(j == 0)
    def _init():
        o_ref[...] = contrib

    @pl.when(j > 0)
    def _acc():
        o_ref[...] += contrib


def kernel(x, W_gate, W_up, W_down):
    return pl.pallas_call(
        _mlp_kernel,
        grid=(_INTER // _BJ,),
        in_specs=[
            pl.BlockSpec((_TOKENS, _HIDDEN), lambda j: (0, 0)),
            pl.BlockSpec((_HIDDEN, _BJ), lambda j: (0, j)),
            pl.BlockSpec((_HIDDEN, _BJ), lambda j: (0, j)),
            pl.BlockSpec((_BJ, _HIDDEN), lambda j: (j, 0)),
        ],
        out_specs=pl.BlockSpec((_TOKENS, _HIDDEN), lambda j: (0, 0)),
        out_shape=jax.ShapeDtypeStruct((_TOKENS, _HIDDEN), jnp.float32),
    )(x, W_gate, W_up, W_down)
